# baseline (speedup 1.0000x reference)
import jax
import jax.numpy as jnp
from jax.experimental import pallas as pl
from jax.experimental.pallas import tpu as pltpu

_CONTRACT_LAST = (((1,), (1,)), ((), ()))


def _mm_accum_kernel(x_ref, w_ref, o_ref):
    @pl.when(pl.program_id(2) == 0)
    def _():
        o_ref[...] = jnp.zeros_like(o_ref)

    o_ref[...] = jax.lax.dot_general(
        x_ref[...], w_ref[...],
        dimension_numbers=_CONTRACT_LAST,
        preferred_element_type=jnp.float32,
    ) + o_ref[...]


def _linear_fused(x2d, w, tm, tn, tk):
    M, K = x2d.shape
    N = w.shape[0]
    grid = (N // tn, M // tm, K // tk)
    out = pl.pallas_call(
        _mm_accum_kernel,
        out_shape=jax.ShapeDtypeStruct((M, N), jnp.float32),
        grid=grid,
        in_specs=[
            pl.BlockSpec((tm, tk), lambda j, i, k: (i, k)),
            pl.BlockSpec((tn, tk), lambda j, i, k: (j, k)),
        ],
        out_specs=pl.BlockSpec((tm, tn), lambda j, i, k: (i, j)),
        compiler_params=pltpu.CompilerParams(
            dimension_semantics=("parallel", "parallel", "arbitrary"),
            vmem_limit_bytes=60 << 20,
        ),
        cost_estimate=pl.CostEstimate(
            flops=2 * M * N * K,
            bytes_accessed=(M * K + N * K) * 4 + M * N * 4,
            transcendentals=0,
        ),
    )(x2d, w)
    return out


def kernel(x, weight):
    orig_lead = x.shape[:-1]
    K = x.shape[-1]
    N = weight.shape[0]
    x2d = x.reshape(-1, K)
    out = _linear_fused(x2d, weight, tm=2048, tn=2048, tk=512)
    return out.reshape(*orig_lead, N)


# speedup vs baseline: 1.0027x; 1.0027x over previous
import jax
import jax.numpy as jnp
from jax.experimental import pallas as pl
from jax.experimental.pallas import tpu as pltpu

_CONTRACT_LAST = (((1,), (1,)), ((), ()))


def _mm_accum_kernel(x_ref, w_ref, o_ref):
    @pl.when(pl.program_id(2) == 0)
    def _():
        o_ref[...] = jnp.zeros_like(o_ref)

    o_ref[...] += jax.lax.dot_general(
        x_ref[...], w_ref[...],
        dimension_numbers=_CONTRACT_LAST,
        preferred_element_type=jnp.float32,
    )


def _linear_fused(x2d, w, tm, tn, tk):
    M, K = x2d.shape
    N = w.shape[0]
    grid = (N // tn, M // tm, K // tk)
    out = pl.pallas_call(
        _mm_accum_kernel,
        out_shape=jax.ShapeDtypeStruct((M, N), jnp.float32),
        grid=grid,
        in_specs=[
            pl.BlockSpec((tm, tk), lambda j, i, k: (i, k)),
            pl.BlockSpec((tn, tk), lambda j, i, k: (j, k)),
        ],
        out_specs=pl.BlockSpec((tm, tn), lambda j, i, k: (i, j)),
        compiler_params=pltpu.CompilerParams(
            dimension_semantics=("parallel", "parallel", "arbitrary"),
            vmem_limit_bytes=60 << 20,
        ),
        cost_estimate=pl.CostEstimate(
            flops=2 * M * N * K,
            bytes_accessed=(M * K + N * K) * 4 + M * N * 4,
            transcendentals=0,
        ),
    )(x2d, w)
    return out


def kernel(x, weight):
    orig_lead = x.shape[:-1]
    K = x.shape[-1]
    N = weight.shape[0]
    x2d = x.reshape(-1, K)
    out = _linear_fused(x2d, weight, tm=2048, tn=2048, tk=512)
    return out.reshape(*orig_lead, N)
